# baseline (speedup 1.0000x reference)
import functools

import jax
import jax.numpy as jnp
from jax import lax
from jax.experimental import pallas as pl
from jax.experimental.pallas import tpu as pltpu
from jax.experimental.pallas import tpu_sc as plsc

DIM = 16
CHUNK = 128


@functools.lru_cache(maxsize=None)
def _sc_gather(B: int, V: int, S: int):
    info = plsc.get_sparse_core_info()
    nw = info.num_cores * info.num_subcores
    assert B % (8 * nw) == 0
    bpw = B // nw
    n_chunks = bpw // CHUNK
    mesh = plsc.VectorSubcoreMesh(core_axis_name="c", subcore_axis_name="s")

    @functools.partial(
        pl.kernel,
        out_type=[
            jax.ShapeDtypeStruct((B, DIM), jnp.float32),
            jax.ShapeDtypeStruct((B, DIM), jnp.float32),
            jax.ShapeDtypeStruct((B, 1), jnp.float32),
            jax.ShapeDtypeStruct((S, DIM), jnp.float32),
            jax.ShapeDtypeStruct((S, 1), jnp.float32),
        ],
        mesh=mesh,
        scratch_types=[
            pltpu.VMEM((B // 32,), jnp.int32),
            pltpu.VMEM((B // 32,), jnp.int32),
            pltpu.VMEM((B // 32, DIM), jnp.float32),
            pltpu.VMEM((B // 32, DIM), jnp.float32),
            pltpu.VMEM((B // 32, 1), jnp.float32),
            pltpu.VMEM((S,), jnp.int32),
            pltpu.VMEM((S, DIM), jnp.float32),
            pltpu.VMEM((S, 1), jnp.float32),
            pltpu.SemaphoreType.DMA,
        ],
        compiler_params=pltpu.CompilerParams(use_tc_tiling_on_sc=False),
    )
    def k(x_hbm, lbl_hbm, emb_hbm, w_hbm, b2_hbm, sid_hbm,
          embedd_o, truew_o, trueb_o, sampw_o, sampb_o,
          xidx_v, lidx_v, erows_v, wrows_v, bvals_v, sidx_v, srows_v, sb_v,
          sem):
        wid = lax.axis_index("s") * info.num_cores + lax.axis_index("c")
        base = wid * bpw
        pltpu.sync_copy(x_hbm.at[pl.ds(base, bpw)], xidx_v)
        pltpu.sync_copy(lbl_hbm.at[pl.ds(base, bpw)], lidx_v)
        cps = []
        for j in range(n_chunks):
            sl = pl.ds(j * CHUNK, CHUNK)
            cps.append(pltpu.async_copy(emb_hbm.at[xidx_v.at[sl]],
                                        erows_v.at[sl], sem))
            cps.append(pltpu.async_copy(w_hbm.at[lidx_v.at[sl]],
                                        wrows_v.at[sl], sem))
            cps.append(pltpu.async_copy(b2_hbm.at[lidx_v.at[sl]],
                                        bvals_v.at[sl], sem))
        for cp in cps:
            cp.wait()
        pltpu.sync_copy(erows_v, embedd_o.at[pl.ds(base, bpw)])
        pltpu.sync_copy(wrows_v, truew_o.at[pl.ds(base, bpw)])
        pltpu.sync_copy(bvals_v, trueb_o.at[pl.ds(base, bpw)])

        @pl.when(wid == 0)
        def _():
            pltpu.sync_copy(sid_hbm, sidx_v)
            cp1 = pltpu.async_copy(w_hbm.at[sidx_v], srows_v, sem)
            cp2 = pltpu.async_copy(b2_hbm.at[sidx_v], sb_v, sem)
            cp1.wait()
            cp2.wait()
            pltpu.sync_copy(srows_v, sampw_o)
            pltpu.sync_copy(sb_v, sampb_o)

    return k


def _tc_loss_body(V, S, e_ref, tw_ref, tb_ref, lbl_ref, sw_ref, sb_ref,
                  sid_ref, out_ref):
    e = e_ref[...]
    tw = tw_ref[...]
    tb = tb_ref[...][:, 0]
    lbl = lbl_ref[...][:, 0]
    sw = sw_ref[...]
    sb = sb_ref[...][:, 0]
    sid = sid_ref[...][:, 0]

    log_vp1 = jnp.log(float(V) + 1.0)

    def log_k_prob(ids):
        idsf = ids.astype(jnp.float32)
        p = (jnp.log(idsf + 2.0) - jnp.log(idsf + 1.0)) / log_vp1
        return jnp.log(float(S) * p)

    def sce(logits, label):
        return (jnp.maximum(logits, 0.0) - logits * label
                + jnp.log1p(jnp.exp(-jnp.abs(logits))))

    true_logits = jnp.sum(e * tw, axis=1) + tb - log_k_prob(lbl)
    sampled = lax.dot_general(e, sw, (((1,), (1,)), ((), ())),
                              preferred_element_type=jnp.float32)
    sampled = sampled + (sb - log_k_prob(sid))[None, :]
    loss = sce(true_logits, 1.0) + jnp.sum(sce(sampled, 0.0), axis=1)
    out_ref[...] = loss[:, None]


@functools.lru_cache(maxsize=None)
def _tc_loss(B: int, V: int, S: int):
    BB = 2048
    return pl.pallas_call(
        functools.partial(_tc_loss_body, V, S),
        grid=(B // BB,),
        in_specs=[
            pl.BlockSpec((BB, DIM), lambda i: (i, 0)),
            pl.BlockSpec((BB, DIM), lambda i: (i, 0)),
            pl.BlockSpec((BB, 1), lambda i: (i, 0)),
            pl.BlockSpec((BB, 1), lambda i: (i, 0)),
            pl.BlockSpec((S, DIM), lambda i: (0, 0)),
            pl.BlockSpec((S, 1), lambda i: (0, 0)),
            pl.BlockSpec((S, 1), lambda i: (0, 0)),
        ],
        out_specs=pl.BlockSpec((BB, 1), lambda i: (i, 0)),
        out_shape=jax.ShapeDtypeStruct((B, 1), jnp.float32),
    )


def kernel(x, target, emb_table, w_score, b_score, sampled_ids):
    B = x.shape[0]
    V, _ = emb_table.shape
    S = sampled_ids.shape[0]
    labels = target.reshape(B)
    b2 = b_score.reshape(V, 1)
    embedd, true_w, true_b, sampled_w, sampled_b = _sc_gather(B, V, S)(
        x, labels, emb_table, w_score, b2, sampled_ids)
    loss = _tc_loss(B, V, S)(
        embedd, true_w, true_b, labels.reshape(B, 1),
        sampled_w, sampled_b, sampled_ids.reshape(S, 1))
    return loss.reshape(B)


# speedup vs baseline: 10.8592x; 10.8592x over previous
import functools

import jax
import jax.numpy as jnp
from jax import lax
from jax.experimental import pallas as pl
from jax.experimental.pallas import tpu as pltpu
from jax.experimental.pallas import tpu_sc as plsc

DIM = 16
CHUNK = 128
GRP = 16


@functools.lru_cache(maxsize=None)
def _sc_gather(B: int, V: int, S: int):
    info = plsc.get_sparse_core_info()
    nw = info.num_cores * info.num_subcores
    assert B % (8 * nw) == 0
    bpw = B // nw
    mesh = plsc.VectorSubcoreMesh(core_axis_name="c", subcore_axis_name="s")

    @functools.partial(
        pl.kernel,
        out_type=[
            jax.ShapeDtypeStruct((DIM, B), jnp.float32),
            jax.ShapeDtypeStruct((DIM, B), jnp.float32),
            jax.ShapeDtypeStruct((B,), jnp.float32),
            jax.ShapeDtypeStruct((DIM, S), jnp.float32),
            jax.ShapeDtypeStruct((S,), jnp.float32),
        ],
        mesh=mesh,
        scratch_types=[
            pltpu.VMEM((bpw,), jnp.int32),
            pltpu.VMEM((bpw,), jnp.int32),
            pltpu.VMEM((bpw,), jnp.int32),
            pltpu.VMEM((GRP, DIM, 128), jnp.float32),
            pltpu.VMEM((GRP, DIM, 128), jnp.float32),
            pltpu.VMEM((DIM, bpw), jnp.float32),
            pltpu.VMEM((DIM, bpw), jnp.float32),
            pltpu.VMEM((bpw,), jnp.float32),
            pltpu.VMEM((S,), jnp.int32),
            pltpu.VMEM((S,), jnp.int32),
            pltpu.VMEM((DIM, S), jnp.float32),
            pltpu.VMEM((S,), jnp.float32),
            pltpu.SemaphoreType.DMA,
            pltpu.SemaphoreType.DMA,
        ],
        compiler_params=pltpu.CompilerParams(needs_layout_passes=False),
    )
    def k(x_hbm, lbl_hbm, embt_hbm, wt_hbm, b_hbm, sid_hbm,
          et_o, wt_o, tb_o, swt_o, sb_o,
          xs_s, ls_s, lidx_v, ecol_v, wcol_v, et_v, wt_v, bv_v,
          sid_s, sidx_v, swt_v, sb_v, sem, semb):
        wid = lax.axis_index("s") * info.num_cores + lax.axis_index("c")
        base = wid * bpw
        pltpu.sync_copy(x_hbm.at[pl.ds(base, bpw)], xs_s)
        pltpu.sync_copy(lbl_hbm.at[pl.ds(base, bpw)], ls_s)
        pltpu.sync_copy(lbl_hbm.at[pl.ds(base, bpw)], lidx_v)

        bcps = [pltpu.async_copy(b_hbm.at[lidx_v.at[pl.ds(j * CHUNK, CHUNK)]],
                                 bv_v.at[pl.ds(j * CHUNK, CHUNK)], semb)
                for j in range(bpw // CHUNK)]

        iota16 = lax.iota(jnp.int32, 16)
        n_grp = bpw // GRP

        def body(g, carry):
            del carry
            e0 = g * GRP
            xvec = xs_s[pl.ds(e0, GRP)]
            lvec = ls_s[pl.ds(e0, GRP)]
            cps = []
            for u in range(GRP):
                xc = pl.multiple_of((xvec[u] >> 7) * 128, 128)
                lc = pl.multiple_of((lvec[u] >> 7) * 128, 128)
                cps.append(pltpu.async_copy(
                    embt_hbm.at[:, pl.ds(xc, 128)], ecol_v.at[u], sem))
                cps.append(pltpu.async_copy(
                    wt_hbm.at[:, pl.ds(lc, 128)], wcol_v.at[u], sem))
            for cp in cps:
                cp.wait()
            for u in range(GRP):
                xlane = jnp.full((16,), xvec[u] & 127, jnp.int32)
                llane = jnp.full((16,), lvec[u] & 127, jnp.int32)
                ecol = e0 + u + jnp.zeros((16,), jnp.int32)
                erow = plsc.load_gather(ecol_v.at[u], [iota16, xlane])
                wrow = plsc.load_gather(wcol_v.at[u], [iota16, llane])
                plsc.store_scatter(et_v, [iota16, ecol], erow)
                plsc.store_scatter(wt_v, [iota16, ecol], wrow)
            return 0

        lax.fori_loop(0, n_grp, body, 0)

        for cp in bcps:
            cp.wait()
        pltpu.sync_copy(et_v, et_o.at[:, pl.ds(base, bpw)])
        pltpu.sync_copy(wt_v, wt_o.at[:, pl.ds(base, bpw)])
        pltpu.sync_copy(bv_v, tb_o.at[pl.ds(base, bpw)])

        @pl.when(wid == 0)
        def _():
            pltpu.sync_copy(sid_hbm, sid_s)
            pltpu.sync_copy(sid_hbm, sidx_v)
            scps = [pltpu.async_copy(b_hbm.at[sidx_v], sb_v, semb)]
            for h in range(S // 16):
                svec = sid_s[pl.ds(h * 16, 16)]
                for u in range(16):
                    sc = pl.multiple_of((svec[u] >> 7) * 128, 128)
                    cp = pltpu.async_copy(
                        wt_hbm.at[:, pl.ds(sc, 128)], wcol_v.at[u], sem)
                    scps.append(cp)
                for cp in scps[1:]:
                    cp.wait()
                del scps[1:]
                for u in range(16):
                    slane = jnp.full((16,), svec[u] & 127, jnp.int32)
                    scol = h * 16 + u + jnp.zeros((16,), jnp.int32)
                    srow = plsc.load_gather(wcol_v.at[u], [iota16, slane])
                    plsc.store_scatter(swt_v, [iota16, scol], srow)
            for cp in scps:
                cp.wait()
            pltpu.sync_copy(swt_v, swt_o)
            pltpu.sync_copy(sb_v, sb_o)

    return k


def _tc_loss_body(V, S, et_ref, wt_ref, tb_ref, lbl_ref, swt_ref, sb_ref,
                  sid_ref, out_ref):
    e = et_ref[...]
    tw = wt_ref[...]
    tb = tb_ref[...]
    lbl = lbl_ref[...]
    swt = swt_ref[...]
    sb = sb_ref[...]
    sid = sid_ref[...]

    log_vp1 = jnp.log(float(V) + 1.0)

    def log_k_prob(ids):
        idsf = ids.astype(jnp.float32)
        p = (jnp.log(idsf + 2.0) - jnp.log(idsf + 1.0)) / log_vp1
        return jnp.log(float(S) * p)

    def sce(logits, label):
        return (jnp.maximum(logits, 0.0) - logits * label
                + jnp.log1p(jnp.exp(-jnp.abs(logits))))

    true_logits = jnp.sum(e * tw, axis=0) + tb - log_k_prob(lbl)
    sampled = lax.dot_general(e, swt, (((0,), (0,)), ((), ())),
                              preferred_element_type=jnp.float32)
    sampled = sampled + (sb - log_k_prob(sid))[None, :]
    loss = sce(true_logits, 1.0) + jnp.sum(sce(sampled, 0.0), axis=1)
    out_ref[...] = loss


@functools.lru_cache(maxsize=None)
def _tc_loss(B: int, V: int, S: int):
    BB = 4096
    return pl.pallas_call(
        functools.partial(_tc_loss_body, V, S),
        grid=(B // BB,),
        in_specs=[
            pl.BlockSpec((DIM, BB), lambda i: (0, i)),
            pl.BlockSpec((DIM, BB), lambda i: (0, i)),
            pl.BlockSpec((BB,), lambda i: (i,)),
            pl.BlockSpec((BB,), lambda i: (i,)),
            pl.BlockSpec((DIM, S), lambda i: (0, 0)),
            pl.BlockSpec((S,), lambda i: (0,)),
            pl.BlockSpec((S,), lambda i: (0,)),
        ],
        out_specs=pl.BlockSpec((BB,), lambda i: (i,)),
        out_shape=jax.ShapeDtypeStruct((B,), jnp.float32),
    )


def kernel(x, target, emb_table, w_score, b_score, sampled_ids):
    B = x.shape[0]
    V, _ = emb_table.shape
    S = sampled_ids.shape[0]
    labels = target.reshape(B)
    et, wt, tb, swt, sb = _sc_gather(B, V, S)(
        x, labels, emb_table.T, w_score.T, b_score, sampled_ids)
    loss = _tc_loss(B, V, S)(et, wt, tb, labels, swt, sb, sampled_ids)
    return loss


# speedup vs baseline: 11.6450x; 1.0724x over previous
import functools

import jax
import jax.numpy as jnp
from jax import lax
from jax.experimental import pallas as pl
from jax.experimental.pallas import tpu as pltpu
from jax.experimental.pallas import tpu_sc as plsc

DIM = 16
CHUNK = 128
GRP = 8


@functools.lru_cache(maxsize=None)
def _sc_gather(B: int, V: int, S: int):
    info = plsc.get_sparse_core_info()
    nw = info.num_cores * info.num_subcores
    assert B % (8 * nw) == 0
    bpw = B // nw
    mesh = plsc.VectorSubcoreMesh(core_axis_name="c", subcore_axis_name="s")

    @functools.partial(
        pl.kernel,
        out_type=[
            jax.ShapeDtypeStruct((DIM, B), jnp.float32),
            jax.ShapeDtypeStruct((DIM, B), jnp.float32),
            jax.ShapeDtypeStruct((B,), jnp.float32),
            jax.ShapeDtypeStruct((DIM, S), jnp.float32),
            jax.ShapeDtypeStruct((S,), jnp.float32),
        ],
        mesh=mesh,
        scratch_types=[
            pltpu.VMEM((bpw,), jnp.int32),
            pltpu.VMEM((bpw,), jnp.int32),
            pltpu.VMEM((bpw,), jnp.int32),
            pltpu.VMEM((2, GRP, DIM, 128), jnp.float32),
            pltpu.VMEM((2, GRP, DIM, 128), jnp.float32),
            pltpu.VMEM((DIM, bpw), jnp.float32),
            pltpu.VMEM((DIM, bpw), jnp.float32),
            pltpu.VMEM((bpw,), jnp.float32),
            pltpu.VMEM((S,), jnp.int32),
            pltpu.VMEM((S,), jnp.int32),
            pltpu.VMEM((DIM, S), jnp.float32),
            pltpu.VMEM((S,), jnp.float32),
            pltpu.SemaphoreType.DMA,
            pltpu.SemaphoreType.DMA,
            pltpu.SemaphoreType.DMA,
        ],
        compiler_params=pltpu.CompilerParams(needs_layout_passes=False),
    )
    def k(x_hbm, lbl_hbm, embt_hbm, wt_hbm, b_hbm, sid_hbm,
          et_o, wt_o, tb_o, swt_o, sb_o,
          xs_s, ls_s, lidx_v, ecol_v, wcol_v, et_v, wt_v, bv_v,
          sid_s, sidx_v, swt_v, sb_v, sem, semb, semb2):
        wid = lax.axis_index("s") * info.num_cores + lax.axis_index("c")
        base = wid * bpw
        pltpu.sync_copy(x_hbm.at[pl.ds(base, bpw)], xs_s)
        pltpu.sync_copy(lbl_hbm.at[pl.ds(base, bpw)], ls_s)
        pltpu.sync_copy(lbl_hbm.at[pl.ds(base, bpw)], lidx_v)

        bcps = [pltpu.async_copy(b_hbm.at[lidx_v.at[pl.ds(j * CHUNK, CHUNK)]],
                                 bv_v.at[pl.ds(j * CHUNK, CHUNK)], semb)
                for j in range(bpw // CHUNK)]

        iota16 = lax.iota(jnp.int32, 16)
        n_sup = bpw // 16

        def issue(k, half, buf, s):
            xvec = xs_s[pl.ds(k * 16, 16)]
            lvec = ls_s[pl.ds(k * 16, 16)]
            for u in range(GRP):
                xc = pl.multiple_of((xvec[half * GRP + u] >> 7) * 128, 128)
                lc = pl.multiple_of((lvec[half * GRP + u] >> 7) * 128, 128)
                pltpu.async_copy(
                    embt_hbm.at[:, pl.ds(xc, 128)], ecol_v.at[buf, u], s)
                pltpu.async_copy(
                    wt_hbm.at[:, pl.ds(lc, 128)], wcol_v.at[buf, u], s)

        def drain(s):
            for _ in range(2 * GRP):
                pltpu.make_async_copy(
                    embt_hbm.at[:, pl.ds(0, 128)], ecol_v.at[0, 0], s).wait()

        def extract(k, half, buf):
            e0 = k * 16 + half * GRP
            xvec = xs_s[pl.ds(k * 16, 16)]
            lvec = ls_s[pl.ds(k * 16, 16)]
            for u in range(GRP):
                xlane = jnp.full((16,), xvec[half * GRP + u] & 127, jnp.int32)
                llane = jnp.full((16,), lvec[half * GRP + u] & 127, jnp.int32)
                ecol = e0 + u + jnp.zeros((16,), jnp.int32)
                erow = plsc.load_gather(ecol_v.at[buf, u], [iota16, xlane])
                wrow = plsc.load_gather(wcol_v.at[buf, u], [iota16, llane])
                plsc.store_scatter(et_v, [iota16, ecol], erow)
                plsc.store_scatter(wt_v, [iota16, ecol], wrow)

        issue(0, 0, 0, sem)
        issue(0, 1, 1, semb2)

        def body(k, carry):
            del carry
            drain(sem)
            extract(k, 0, 0)

            @pl.when(k + 1 < n_sup)
            def _():
                issue(k + 1, 0, 0, sem)

            drain(semb2)
            extract(k, 1, 1)

            @pl.when(k + 1 < n_sup)
            def _():
                issue(k + 1, 1, 1, semb2)

            return 0

        lax.fori_loop(0, n_sup, body, 0)

        for cp in bcps:
            cp.wait()
        pltpu.sync_copy(et_v, et_o.at[:, pl.ds(base, bpw)])
        pltpu.sync_copy(wt_v, wt_o.at[:, pl.ds(base, bpw)])
        pltpu.sync_copy(bv_v, tb_o.at[pl.ds(base, bpw)])

        @pl.when(wid == 0)
        def _():
            pltpu.sync_copy(sid_hbm, sid_s)
            pltpu.sync_copy(sid_hbm, sidx_v)
            scps = [pltpu.async_copy(b_hbm.at[sidx_v], sb_v, semb)]
            for h in range(S // 16):
                svec = sid_s[pl.ds(h * 16, 16)]
                for half in range(2):
                    for u in range(GRP):
                        sc = pl.multiple_of(
                            (svec[half * GRP + u] >> 7) * 128, 128)
                        cp = pltpu.async_copy(
                            wt_hbm.at[:, pl.ds(sc, 128)],
                            wcol_v.at[half, u], sem)
                        scps.append(cp)
                for cp in scps[1:]:
                    cp.wait()
                del scps[1:]
                for half in range(2):
                    for u in range(GRP):
                        slane = jnp.full(
                            (16,), svec[half * GRP + u] & 127, jnp.int32)
                        scol = (h * 16 + half * GRP + u
                                + jnp.zeros((16,), jnp.int32))
                        srow = plsc.load_gather(
                            wcol_v.at[half, u], [iota16, slane])
                        plsc.store_scatter(swt_v, [iota16, scol], srow)
            for cp in scps:
                cp.wait()
            pltpu.sync_copy(swt_v, swt_o)
            pltpu.sync_copy(sb_v, sb_o)

    return k


def _tc_loss_body(V, S, et_ref, wt_ref, tb_ref, lbl_ref, swt_ref, sb_ref,
                  sid_ref, out_ref):
    e = et_ref[...]
    tw = wt_ref[...]
    tb = tb_ref[...]
    lbl = lbl_ref[...]
    swt = swt_ref[...]
    sb = sb_ref[...]
    sid = sid_ref[...]

    log_vp1 = jnp.log(float(V) + 1.0)

    def log_k_prob(ids):
        idsf = ids.astype(jnp.float32)
        p = (jnp.log(idsf + 2.0) - jnp.log(idsf + 1.0)) / log_vp1
        return jnp.log(float(S) * p)

    def sce(logits, label):
        return (jnp.maximum(logits, 0.0) - logits * label
                + jnp.log1p(jnp.exp(-jnp.abs(logits))))

    true_logits = jnp.sum(e * tw, axis=0) + tb - log_k_prob(lbl)
    sampled = lax.dot_general(e, swt, (((0,), (0,)), ((), ())),
                              preferred_element_type=jnp.float32)
    sampled = sampled + (sb - log_k_prob(sid))[None, :]
    loss = sce(true_logits, 1.0) + jnp.sum(sce(sampled, 0.0), axis=1)
    out_ref[...] = loss


@functools.lru_cache(maxsize=None)
def _tc_loss(B: int, V: int, S: int):
    BB = 4096
    return pl.pallas_call(
        functools.partial(_tc_loss_body, V, S),
        grid=(B // BB,),
        in_specs=[
            pl.BlockSpec((DIM, BB), lambda i: (0, i)),
            pl.BlockSpec((DIM, BB), lambda i: (0, i)),
            pl.BlockSpec((BB,), lambda i: (i,)),
            pl.BlockSpec((BB,), lambda i: (i,)),
            pl.BlockSpec((DIM, S), lambda i: (0, 0)),
            pl.BlockSpec((S,), lambda i: (0,)),
            pl.BlockSpec((S,), lambda i: (0,)),
        ],
        out_specs=pl.BlockSpec((BB,), lambda i: (i,)),
        out_shape=jax.ShapeDtypeStruct((B,), jnp.float32),
    )


def kernel(x, target, emb_table, w_score, b_score, sampled_ids):
    B = x.shape[0]
    V, _ = emb_table.shape
    S = sampled_ids.shape[0]
    labels = target.reshape(B)
    et, wt, tb, swt, sb = _sc_gather(B, V, S)(
        x, labels, emb_table.T, w_score.T, b_score, sampled_ids)
    loss = _tc_loss(B, V, S)(et, wt, tb, labels, swt, sb, sampled_ids)
    return loss


# speedup vs baseline: 11.6460x; 1.0001x over previous
import functools

import jax
import jax.numpy as jnp
from jax import lax
from jax.experimental import pallas as pl
from jax.experimental.pallas import tpu as pltpu
from jax.experimental.pallas import tpu_sc as plsc

DIM = 16
CHUNK = 128
GRP = 8


@functools.lru_cache(maxsize=None)
def _sc_gather(B: int, V: int, S: int):
    info = plsc.get_sparse_core_info()
    nw = info.num_cores * info.num_subcores
    assert B % (8 * nw) == 0
    bpw = B // nw
    mesh = plsc.VectorSubcoreMesh(core_axis_name="c", subcore_axis_name="s")

    @functools.partial(
        pl.kernel,
        out_type=[
            jax.ShapeDtypeStruct((DIM, B), jnp.float32),
            jax.ShapeDtypeStruct((DIM, B), jnp.float32),
            jax.ShapeDtypeStruct((B,), jnp.float32),
            jax.ShapeDtypeStruct((DIM, S), jnp.float32),
            jax.ShapeDtypeStruct((S,), jnp.float32),
        ],
        mesh=mesh,
        scratch_types=[
            pltpu.VMEM((bpw,), jnp.int32),
            pltpu.VMEM((bpw,), jnp.int32),
            pltpu.VMEM((bpw,), jnp.int32),
            pltpu.VMEM((2, GRP, DIM, 128), jnp.float32),
            pltpu.VMEM((2, GRP, DIM, 128), jnp.float32),
            pltpu.VMEM((DIM, bpw), jnp.float32),
            pltpu.VMEM((DIM, bpw), jnp.float32),
            pltpu.VMEM((bpw,), jnp.float32),
            pltpu.VMEM((S,), jnp.int32),
            pltpu.VMEM((S,), jnp.int32),
            pltpu.VMEM((DIM, S), jnp.float32),
            pltpu.VMEM((S,), jnp.float32),
            pltpu.SemaphoreType.DMA,
            pltpu.SemaphoreType.DMA,
            pltpu.SemaphoreType.DMA,
        ],
        compiler_params=pltpu.CompilerParams(needs_layout_passes=False),
    )
    def k(x_hbm, lbl_hbm, embt_hbm, wt_hbm, b_hbm, sid_hbm,
          et_o, wt_o, tb_o, swt_o, sb_o,
          xs_s, ls_s, lidx_v, ecol_v, wcol_v, et_v, wt_v, bv_v,
          sid_s, sidx_v, swt_v, sb_v, sem, semb, semb2):
        wid = lax.axis_index("s") * info.num_cores + lax.axis_index("c")
        base = wid * bpw
        pltpu.sync_copy(x_hbm.at[pl.ds(base, bpw)], xs_s)
        pltpu.sync_copy(lbl_hbm.at[pl.ds(base, bpw)], ls_s)
        pltpu.sync_copy(lbl_hbm.at[pl.ds(base, bpw)], lidx_v)

        bcps = [pltpu.async_copy(b_hbm.at[lidx_v.at[pl.ds(j * CHUNK, CHUNK)]],
                                 bv_v.at[pl.ds(j * CHUNK, CHUNK)], semb)
                for j in range(bpw // CHUNK)]

        iota16 = lax.iota(jnp.int32, 16)
        n_sup = bpw // 16

        def issue(k, half, buf, s):
            xvec = xs_s[pl.ds(k * 16, 16)]
            lvec = ls_s[pl.ds(k * 16, 16)]
            for u in range(GRP):
                xc = pl.multiple_of((xvec[half * GRP + u] >> 7) * 128, 128)
                lc = pl.multiple_of((lvec[half * GRP + u] >> 7) * 128, 128)
                pltpu.async_copy(
                    embt_hbm.at[:, pl.ds(xc, 128)], ecol_v.at[buf, u], s)
                pltpu.async_copy(
                    wt_hbm.at[:, pl.ds(lc, 128)], wcol_v.at[buf, u], s)

        def drain(s):
            for _ in range(2 * GRP):
                pltpu.make_async_copy(
                    embt_hbm.at[:, pl.ds(0, 128)], ecol_v.at[0, 0], s).wait()

        def extract(k, half, buf):
            e0 = k * 16 + half * GRP
            xvec = xs_s[pl.ds(k * 16, 16)]
            lvec = ls_s[pl.ds(k * 16, 16)]
            for u in range(GRP):
                xlane = jnp.full((16,), xvec[half * GRP + u] & 127, jnp.int32)
                llane = jnp.full((16,), lvec[half * GRP + u] & 127, jnp.int32)
                ecol = e0 + u + jnp.zeros((16,), jnp.int32)
                erow = plsc.load_gather(ecol_v.at[buf, u], [iota16, xlane])
                wrow = plsc.load_gather(wcol_v.at[buf, u], [iota16, llane])
                plsc.store_scatter(et_v, [iota16, ecol], erow)
                plsc.store_scatter(wt_v, [iota16, ecol], wrow)

        issue(0, 0, 0, sem)
        issue(0, 1, 1, semb2)

        def body(k, carry):
            del carry
            drain(sem)
            extract(k, 0, 0)

            @pl.when(k + 1 < n_sup)
            def _():
                issue(k + 1, 0, 0, sem)

            drain(semb2)
            extract(k, 1, 1)

            @pl.when(k + 1 < n_sup)
            def _():
                issue(k + 1, 1, 1, semb2)

            return 0

        lax.fori_loop(0, n_sup, body, 0)

        for cp in bcps:
            cp.wait()
        pltpu.sync_copy(et_v, et_o.at[:, pl.ds(base, bpw)])
        pltpu.sync_copy(wt_v, wt_o.at[:, pl.ds(base, bpw)])
        pltpu.sync_copy(bv_v, tb_o.at[pl.ds(base, bpw)])

        @pl.when(wid == 0)
        def _():
            pltpu.sync_copy(sid_hbm, sid_s)
            pltpu.sync_copy(sid_hbm, sidx_v)
            scps = [pltpu.async_copy(b_hbm.at[sidx_v], sb_v, semb)]
            for h in range(S // 16):
                svec = sid_s[pl.ds(h * 16, 16)]
                for half in range(2):
                    for u in range(GRP):
                        sc = pl.multiple_of(
                            (svec[half * GRP + u] >> 7) * 128, 128)
                        cp = pltpu.async_copy(
                            wt_hbm.at[:, pl.ds(sc, 128)],
                            wcol_v.at[half, u], sem)
                        scps.append(cp)
                for cp in scps[1:]:
                    cp.wait()
                del scps[1:]
                for half in range(2):
                    for u in range(GRP):
                        slane = jnp.full(
                            (16,), svec[half * GRP + u] & 127, jnp.int32)
                        scol = (h * 16 + half * GRP + u
                                + jnp.zeros((16,), jnp.int32))
                        srow = plsc.load_gather(
                            wcol_v.at[half, u], [iota16, slane])
                        plsc.store_scatter(swt_v, [iota16, scol], srow)
            for cp in scps:
                cp.wait()
            pltpu.sync_copy(swt_v, swt_o)
            pltpu.sync_copy(sb_v, sb_o)

    return k


def _tc_loss_body(V, S, et_ref, wt_ref, tb_ref, lbl_ref, swt_ref, sb_ref,
                  sid_ref, out_ref):
    e = et_ref[...]
    tw = wt_ref[...]
    tb = tb_ref[...]
    lbl = lbl_ref[...]
    swt = swt_ref[...]
    sb = sb_ref[...]
    sid = sid_ref[...]

    log_vp1 = jnp.log(float(V) + 1.0)

    def log_k_prob(ids):
        idsf = ids.astype(jnp.float32)
        p = (jnp.log(idsf + 2.0) - jnp.log(idsf + 1.0)) / log_vp1
        return jnp.log(float(S) * p)

    def sce(logits, label):
        return (jnp.maximum(logits, 0.0) - logits * label
                + jnp.log1p(jnp.exp(-jnp.abs(logits))))

    true_logits = jnp.sum(e * tw, axis=0) + tb - log_k_prob(lbl)
    sampled = lax.dot_general(e, swt, (((0,), (0,)), ((), ())),
                              preferred_element_type=jnp.float32)
    sampled = sampled + (sb - log_k_prob(sid))[None, :]
    loss = sce(true_logits, 1.0) + jnp.sum(sce(sampled, 0.0), axis=1)
    out_ref[...] = loss


@functools.lru_cache(maxsize=None)
def _tc_loss(B: int, V: int, S: int):
    BB = B
    return pl.pallas_call(
        functools.partial(_tc_loss_body, V, S),
        grid=(B // BB,),
        in_specs=[
            pl.BlockSpec((DIM, BB), lambda i: (0, i)),
            pl.BlockSpec((DIM, BB), lambda i: (0, i)),
            pl.BlockSpec((BB,), lambda i: (i,)),
            pl.BlockSpec((BB,), lambda i: (i,)),
            pl.BlockSpec((DIM, S), lambda i: (0, 0)),
            pl.BlockSpec((S,), lambda i: (0,)),
            pl.BlockSpec((S,), lambda i: (0,)),
        ],
        out_specs=pl.BlockSpec((BB,), lambda i: (i,)),
        out_shape=jax.ShapeDtypeStruct((B,), jnp.float32),
    )


def kernel(x, target, emb_table, w_score, b_score, sampled_ids):
    B = x.shape[0]
    V, _ = emb_table.shape
    S = sampled_ids.shape[0]
    labels = target.reshape(B)
    et, wt, tb, swt, sb = _sc_gather(B, V, S)(
        x, labels, emb_table.T, w_score.T, b_score, sampled_ids)
    loss = _tc_loss(B, V, S)(et, wt, tb, labels, swt, sb, sampled_ids)
    return loss


# speedup vs baseline: 11.6741x; 1.0024x over previous
import functools

import jax
import jax.numpy as jnp
from jax import lax
from jax.experimental import pallas as pl
from jax.experimental.pallas import tpu as pltpu
from jax.experimental.pallas import tpu_sc as plsc

DIM = 16
CHUNK = 128
GRP = 8


@functools.lru_cache(maxsize=None)
def _sc_gather(B: int, V: int, S: int):
    info = plsc.get_sparse_core_info()
    nw = info.num_cores * info.num_subcores
    assert B % (8 * nw) == 0
    bpw = B // nw
    mesh = plsc.VectorSubcoreMesh(core_axis_name="c", subcore_axis_name="s")

    @functools.partial(
        pl.kernel,
        out_type=[
            jax.ShapeDtypeStruct((DIM, B), jnp.float32),
            jax.ShapeDtypeStruct((DIM, B), jnp.float32),
            jax.ShapeDtypeStruct((B,), jnp.float32),
            jax.ShapeDtypeStruct((DIM, S), jnp.float32),
            jax.ShapeDtypeStruct((S,), jnp.float32),
        ],
        mesh=mesh,
        scratch_types=[
            pltpu.VMEM((bpw,), jnp.int32),
            pltpu.VMEM((bpw,), jnp.int32),
            pltpu.VMEM((bpw,), jnp.int32),
            pltpu.VMEM((2, GRP, DIM, 128), jnp.float32),
            pltpu.VMEM((2, GRP, DIM, 128), jnp.float32),
            pltpu.VMEM((DIM, bpw), jnp.float32),
            pltpu.VMEM((DIM, bpw), jnp.float32),
            pltpu.VMEM((bpw,), jnp.float32),
            pltpu.VMEM((S,), jnp.int32),
            pltpu.VMEM((S,), jnp.int32),
            pltpu.VMEM((DIM, S), jnp.float32),
            pltpu.VMEM((S,), jnp.float32),
            pltpu.SemaphoreType.DMA,
            pltpu.SemaphoreType.DMA,
            pltpu.SemaphoreType.DMA,
        ],
        compiler_params=pltpu.CompilerParams(needs_layout_passes=False),
    )
    def k(x_hbm, lbl_hbm, embt_hbm, wt_hbm, b_hbm, sid_hbm,
          et_o, wt_o, tb_o, swt_o, sb_o,
          xs_s, ls_s, lidx_v, ecol_v, wcol_v, et_v, wt_v, bv_v,
          sid_s, sidx_v, swt_v, sb_v, sem, semb, semb2):
        wid = lax.axis_index("s") * info.num_cores + lax.axis_index("c")
        base = wid * bpw
        pltpu.sync_copy(x_hbm.at[pl.ds(base, bpw)], xs_s)
        pltpu.sync_copy(lbl_hbm.at[pl.ds(base, bpw)], ls_s)
        pltpu.sync_copy(lbl_hbm.at[pl.ds(base, bpw)], lidx_v)

        bcps = [pltpu.async_copy(b_hbm.at[lidx_v.at[pl.ds(j * CHUNK, CHUNK)]],
                                 bv_v.at[pl.ds(j * CHUNK, CHUNK)], semb)
                for j in range(bpw // CHUNK)]

        iota16 = lax.iota(jnp.int32, 16)
        n_sup = bpw // 16

        def issue(k, half, buf, s):
            xvec = xs_s[pl.ds(k * 16, 16)]
            lvec = ls_s[pl.ds(k * 16, 16)]
            for u in range(GRP):
                xc = pl.multiple_of((xvec[half * GRP + u] >> 7) * 128, 128)
                lc = pl.multiple_of((lvec[half * GRP + u] >> 7) * 128, 128)
                pltpu.async_copy(
                    embt_hbm.at[:, pl.ds(xc, 128)], ecol_v.at[buf, u], s)
                pltpu.async_copy(
                    wt_hbm.at[:, pl.ds(lc, 128)], wcol_v.at[buf, u], s)

        def drain(s):
            for _ in range(4):
                pltpu.make_async_copy(
                    embt_hbm.at[:, pl.ds(0, bpw)],
                    et_v, s).wait()

        def extract(k, half, buf):
            e0 = k * 16 + half * GRP
            xvec = xs_s[pl.ds(k * 16, 16)]
            lvec = ls_s[pl.ds(k * 16, 16)]
            for u in range(GRP):
                xlane = jnp.full((16,), xvec[half * GRP + u] & 127, jnp.int32)
                llane = jnp.full((16,), lvec[half * GRP + u] & 127, jnp.int32)
                ecol = e0 + u + jnp.zeros((16,), jnp.int32)
                erow = plsc.load_gather(ecol_v.at[buf, u], [iota16, xlane])
                wrow = plsc.load_gather(wcol_v.at[buf, u], [iota16, llane])
                plsc.store_scatter(et_v, [iota16, ecol], erow)
                plsc.store_scatter(wt_v, [iota16, ecol], wrow)

        issue(0, 0, 0, sem)
        issue(0, 1, 1, semb2)

        def body(k, carry):
            del carry
            drain(sem)
            extract(k, 0, 0)

            @pl.when(k + 1 < n_sup)
            def _():
                issue(k + 1, 0, 0, sem)

            drain(semb2)
            extract(k, 1, 1)

            @pl.when(k + 1 < n_sup)
            def _():
                issue(k + 1, 1, 1, semb2)

            return 0

        lax.fori_loop(0, n_sup, body, 0)

        for cp in bcps:
            cp.wait()
        pltpu.sync_copy(et_v, et_o.at[:, pl.ds(base, bpw)])
        pltpu.sync_copy(wt_v, wt_o.at[:, pl.ds(base, bpw)])
        pltpu.sync_copy(bv_v, tb_o.at[pl.ds(base, bpw)])

        @pl.when(wid == 0)
        def _():
            pltpu.sync_copy(sid_hbm, sid_s)
            pltpu.sync_copy(sid_hbm, sidx_v)
            scps = [pltpu.async_copy(b_hbm.at[sidx_v], sb_v, semb)]
            for h in range(S // 16):
                svec = sid_s[pl.ds(h * 16, 16)]
                for half in range(2):
                    for u in range(GRP):
                        sc = pl.multiple_of(
                            (svec[half * GRP + u] >> 7) * 128, 128)
                        cp = pltpu.async_copy(
                            wt_hbm.at[:, pl.ds(sc, 128)],
                            wcol_v.at[half, u], sem)
                        scps.append(cp)
                for cp in scps[1:]:
                    cp.wait()
                del scps[1:]
                for half in range(2):
                    for u in range(GRP):
                        slane = jnp.full(
                            (16,), svec[half * GRP + u] & 127, jnp.int32)
                        scol = (h * 16 + half * GRP + u
                                + jnp.zeros((16,), jnp.int32))
                        srow = plsc.load_gather(
                            wcol_v.at[half, u], [iota16, slane])
                        plsc.store_scatter(swt_v, [iota16, scol], srow)
            for cp in scps:
                cp.wait()
            pltpu.sync_copy(swt_v, swt_o)
            pltpu.sync_copy(sb_v, sb_o)

    return k


def _tc_loss_body(V, S, et_ref, wt_ref, tb_ref, lbl_ref, swt_ref, sb_ref,
                  sid_ref, out_ref):
    e = et_ref[...]
    tw = wt_ref[...]
    tb = tb_ref[...]
    lbl = lbl_ref[...]
    swt = swt_ref[...]
    sb = sb_ref[...]
    sid = sid_ref[...]

    log_vp1 = jnp.log(float(V) + 1.0)

    def log_k_prob(ids):
        idsf = ids.astype(jnp.float32)
        p = (jnp.log(idsf + 2.0) - jnp.log(idsf + 1.0)) / log_vp1
        return jnp.log(float(S) * p)

    def sce(logits, label):
        return (jnp.maximum(logits, 0.0) - logits * label
                + jnp.log1p(jnp.exp(-jnp.abs(logits))))

    true_logits = jnp.sum(e * tw, axis=0) + tb - log_k_prob(lbl)
    sampled = lax.dot_general(e, swt, (((0,), (0,)), ((), ())),
                              preferred_element_type=jnp.float32)
    sampled = sampled + (sb - log_k_prob(sid))[None, :]
    loss = sce(true_logits, 1.0) + jnp.sum(sce(sampled, 0.0), axis=1)
    out_ref[...] = loss


@functools.lru_cache(maxsize=None)
def _tc_loss(B: int, V: int, S: int):
    BB = B
    return pl.pallas_call(
        functools.partial(_tc_loss_body, V, S),
        grid=(B // BB,),
        in_specs=[
            pl.BlockSpec((DIM, BB), lambda i: (0, i)),
            pl.BlockSpec((DIM, BB), lambda i: (0, i)),
            pl.BlockSpec((BB,), lambda i: (i,)),
            pl.BlockSpec((BB,), lambda i: (i,)),
            pl.BlockSpec((DIM, S), lambda i: (0, 0)),
            pl.BlockSpec((S,), lambda i: (0,)),
            pl.BlockSpec((S,), lambda i: (0,)),
        ],
        out_specs=pl.BlockSpec((BB,), lambda i: (i,)),
        out_shape=jax.ShapeDtypeStruct((B,), jnp.float32),
    )


def kernel(x, target, emb_table, w_score, b_score, sampled_ids):
    B = x.shape[0]
    V, _ = emb_table.shape
    S = sampled_ids.shape[0]
    labels = target.reshape(B)
    et, wt, tb, swt, sb = _sc_gather(B, V, S)(
        x, labels, emb_table.T, w_score.T, b_score, sampled_ids)
    loss = _tc_loss(B, V, S)(et, wt, tb, labels, swt, sb, sampled_ids)
    return loss
